# baseline (speedup 1.0000x reference)
import functools

import jax
import jax.numpy as jnp
from jax import lax
from jax.experimental import pallas as pl
from jax.experimental.pallas import tpu as pltpu
from jax.experimental.pallas import tpu_sc as plsc

NUM_NODES = {'paper': 100000, 'author': 100000, 'institution': 8740, 'field_of_study': 59965}
EDGE_TYPES = [('author', 'writes', 'paper'), ('paper', 'cites', 'paper'), ('paper', 'has_topic', 'field_of_study'), ('author', 'affiliated_with', 'institution'), ('institution', 'employs', 'author'), ('field_of_study', 'rev_topic', 'paper'), ('paper', 'written_by', 'author'), ('paper', 'referenced_by', 'paper')]
HIDDEN = 256
HEADS = 4
DH = HIDDEN // HEADS
E = 100000

NTILES = 16
NCORES = 2
ES = 6256
E_PAD = NTILES * ES
NB = ES // 16
B = 48
R = 2560
SR = R // NTILES
SEG = 512
EST = ES + B
NSLAB = NCORES * NTILES * EST + 8


def _seg_kernel(nch, xs_hbm, src_hbm, dst_hbm, asf_hbm, adf_hbm, amax_hbm,
                out_hbm, stage_hbm, stgd_hbm,
                srcb, dstb, clid,
                ibatch, mbatch, gbuf, sbuf, dbuf, asq, adq, sdidx, ddidx,
                amaxb, cntw, acc,
                mbuf, cbuf, gidx, dgidx, abuf, dabuf, cntb,
                meta_sh, cnt_sh, gsem, asem, bsem):
    c = lax.axis_index("c")
    s = lax.axis_index("s")
    h_pad = nch * R
    sbase = (c * NTILES + s) * EST

    pltpu.sync_copy(src_hbm.at[pl.ds(pl.multiple_of(s * ES, 8), ES)], srcb.at[pl.ds(0, ES)])
    pltpu.sync_copy(dst_hbm.at[pl.ds(pl.multiple_of(s * ES, 8), ES)], dstb)
    pltpu.sync_copy(amax_hbm, amaxb)

    zv = jnp.zeros((16,), jnp.float32)
    zi = jnp.zeros((16,), jnp.int32)
    srcb[pl.ds(ES, 16)] = zi

    lanes = lax.iota(jnp.int32, 16)
    qh = jnp.bitwise_and(lanes, 3)
    qe = jnp.right_shift(lanes, 2)
    amaxq = plsc.load_gather(amaxb, [qh])

    def chunk_body(ci, _):
        lo = c * h_pad + ci * R

        def comp_body(b, cnt):
            off = b * 16
            dl = dstb[pl.ds(off, 16)] - lo
            m = (dl >= 0) & (dl < R)
            cum = plsc.cumsum(jnp.where(m, 1, 0))
            plsc.store_scatter(clid, [cnt + cum - 1], off + lanes, mask=m)
            return cnt + jnp.max(cum)

        cnt = lax.fori_loop(0, NB, comp_body, jnp.int32(0))
        padv = jnp.full((16,), ES, jnp.int32)
        for j in range(B // 16):
            clid[pl.ds(cnt + j * 16, 16)] = padv

        nbat = (cnt + (B - 1)) // B

        def batch_body(k, _):
            base = k * B
            for j in range(B // 16):
                lidv = clid[pl.ds(base + j * 16, 16)]
                ibatch[pl.ds(j * 16, 16)] = plsc.load_gather(srcb, [lidv])
                ov = plsc.load_gather(dstb, [jnp.minimum(lidv, ES - 1)]) - lo
                mbatch[pl.ds(j * 16, 16)] = (base + j * 16 + lanes) * 4096 + ov
            for j2 in range(B // 4):
                lidq = plsc.load_gather(clid, [base + j2 * 4 + qe])
                srcq = plsc.load_gather(srcb, [jnp.minimum(lidq, ES)])
                sdidx[pl.ds(j2 * 16, 16)] = srcq * 4 + qh
                dstq = plsc.load_gather(dstb, [jnp.minimum(lidq, ES - 1)])
                ddidx[pl.ds(j2 * 16, 16)] = dstq * 4 + qh
            cp1 = pltpu.async_copy(asf_hbm.at[sdidx], asq, asem)
            cp2 = pltpu.async_copy(adf_hbm.at[ddidx], adq, bsem)
            cp3 = pltpu.async_copy(xs_hbm.at[ibatch], gbuf, gsem)
            cp1.wait()
            cp2.wait()
            for j2 in range(B * 4 // 16):
                av = asq[pl.ds(j2 * 16, 16)] + adq[pl.ds(j2 * 16, 16)]
                ev_ = jnp.maximum(av, 0.0) + 0.2 * jnp.minimum(av, 0.0)
                mv = adq[pl.ds(j2 * 16, 16)] + amaxq
                mb_ = jnp.maximum(mv, 0.0) + 0.2 * jnp.minimum(mv, 0.0)
                dbuf[pl.ds(j2 * 16, 16)] = jnp.exp(ev_ - mb_)
            cp3.wait()

            def edge_body(i, _):
                for r in (4 * i, 4 * i + 1, 4 * i + 2, 4 * i + 3):
                    idxv = jnp.full((16,), 4 * r, jnp.int32)
                    ev = [plsc.load_gather(dbuf, [idxv + h]) for h in range(4)]
                    for j in range(16):
                        sbuf[r, pl.ds(j * 16, 16)] = gbuf[r, pl.ds(j * 16, 16)] * ev[j // 4]
                return 0

            lax.fori_loop(0, B // 4, edge_body, 0)
            pltpu.sync_copy(sbuf, stage_hbm.at[pl.ds(pl.multiple_of(sbase + base, 8), B)])
            pltpu.sync_copy(dbuf, stgd_hbm.at[pl.ds(pl.multiple_of(4 * (sbase + base), 8), 4 * B)])
            pltpu.sync_copy(mbatch, meta_sh.at[pl.ds(pl.multiple_of(s * (ES + SEG) + base, 8), B)])
            return 0

        lax.fori_loop(0, nbat, batch_body, 0)
        cntw[pl.ds(0, 16)] = jnp.full((16,), cnt, jnp.int32)
        pltpu.sync_copy(cntw, cnt_sh.at[pl.ds(pl.multiple_of(s * 16, 8), 16)])
        plsc.subcore_barrier()

        own_lo = s * SR

        def zrow_body(r2, _):
            for j in range(17):
                acc[r2, pl.ds(j * 16, 16)] = zv
            return 0

        lax.fori_loop(0, SR, zrow_body, 0)
        pltpu.sync_copy(cnt_sh, cntb)
        wtail = jnp.full((16,), (s + 1) * SR, jnp.int32)

        def src_tile_body(s2, _):
            cnt2 = jnp.max(cntb[pl.ds(s2 * 16, 16)])
            gb = (c * NTILES + s2) * EST

            def seg_cond(st):
                return st < cnt2

            def seg_body(st):
                pltpu.sync_copy(meta_sh.at[pl.ds(pl.multiple_of(s2 * (ES + SEG) + st, 8), SEG)], mbuf.at[pl.ds(0, SEG)])
                n = jnp.minimum(cnt2 - st, SEG)

                def blk(b2, cc):
                    w = mbuf[pl.ds(b2 * 16, 16)]
                    dl = jnp.bitwise_and(w, 4095)
                    ok = ((b2 * 16 + lanes) < n) & (dl >= own_lo) & (dl < own_lo + SR)
                    cum = plsc.cumsum(jnp.where(ok, 1, 0))
                    plsc.store_scatter(cbuf, [cc + cum - 1], w, mask=ok)
                    return cc + jnp.max(cum)

                cc = lax.fori_loop(0, SEG // 16, blk, jnp.int32(0))
                for j in range(B // 16):
                    cbuf[pl.ds(cc + j * 16, 16)] = wtail
                nb2 = (cc + (B - 1)) // B

                def b2_body(k2, _):
                    for j in range(B // 16):
                        wv = cbuf[pl.ds(k2 * B + j * 16, 16)]
                        gidx[pl.ds(j * 16, 16)] = gb + jnp.right_shift(wv, 12)
                    for j2 in range(B // 4):
                        wq = plsc.load_gather(
                            cbuf, [k2 * B + j2 * 4 + jnp.right_shift(lanes, 2)])
                        dgidx[pl.ds(j2 * 16, 16)] = (
                            (gb + jnp.right_shift(wq, 12)) * 4 + jnp.bitwise_and(lanes, 3))
                    pltpu.async_copy(stage_hbm.at[gidx], abuf, asem).wait()
                    pltpu.async_copy(stgd_hbm.at[dgidx], dabuf, asem).wait()

                    def erow(i3, _):
                        for r3 in (4 * i3, 4 * i3 + 1, 4 * i3 + 2, 4 * i3 + 3):
                            wv = plsc.load_gather(cbuf, [jnp.full((16,), k2 * B + r3, jnp.int32)])
                            dlr = jnp.max(jnp.bitwise_and(wv, 4095)) - own_lo
                            for j in range(16):
                                acc[dlr, pl.ds(j * 16, 16)] = (
                                    acc[dlr, pl.ds(j * 16, 16)] + abuf[r3, pl.ds(j * 16, 16)])
                            dg = plsc.load_gather(dabuf, [r3 * 4 + jnp.minimum(lanes, 3)])
                            dv = jnp.where(lanes < 4, dg, zv)
                            acc[dlr, pl.ds(256, 16)] = acc[dlr, pl.ds(256, 16)] + dv
                        return 0

                    lax.fori_loop(0, B // 4, erow, 0)
                    return 0

                lax.fori_loop(0, nb2, b2_body, 0)
                return st + SEG

            lax.while_loop(seg_cond, seg_body, jnp.int32(0))
            return 0

        lax.fori_loop(0, NTILES, src_tile_body, 0)

        pltpu.sync_copy(acc.at[pl.ds(0, SR)], out_hbm.at[pl.ds(pl.multiple_of(lo + own_lo, 8), SR)])
        plsc.subcore_barrier()
        return 0

    lax.fori_loop(0, nch, chunk_body, 0)


@functools.partial(jax.jit, static_argnums=(6,))
def _sc_aggregate(xs, src, dst, asf, adf, amax16, nch):
    h_pad = nch * R
    mesh = plsc.VectorSubcoreMesh(core_axis_name="c", subcore_axis_name="s")
    f = pl.kernel(
        functools.partial(_seg_kernel, nch),
        compiler_params=pltpu.CompilerParams(needs_layout_passes=False),
        out_type=(
            jax.ShapeDtypeStruct((2 * h_pad, 272), jnp.float32),
            jax.ShapeDtypeStruct((NSLAB, 256), jnp.float32),
            jax.ShapeDtypeStruct((4 * NSLAB,), jnp.float32),
        ),
        mesh=mesh,
        scratch_types=[
            pltpu.VMEM((ES + 16,), jnp.int32),
            pltpu.VMEM((ES,), jnp.int32),
            pltpu.VMEM((ES + B,), jnp.int32),
            pltpu.VMEM((B,), jnp.int32),
            pltpu.VMEM((B,), jnp.int32),
            pltpu.VMEM((B, 256), jnp.float32),
            pltpu.VMEM((B, 256), jnp.float32),
            pltpu.VMEM((4 * B,), jnp.float32),
            pltpu.VMEM((4 * B,), jnp.float32),
            pltpu.VMEM((4 * B,), jnp.float32),
            pltpu.VMEM((4 * B,), jnp.int32),
            pltpu.VMEM((4 * B,), jnp.int32),
            pltpu.VMEM((16,), jnp.float32),
            pltpu.VMEM((16,), jnp.int32),
            pltpu.VMEM((SR + 8, 272), jnp.float32),
            pltpu.VMEM((SEG + 16,), jnp.int32),
            pltpu.VMEM((SEG + B + 16,), jnp.int32),
            pltpu.VMEM((B,), jnp.int32),
            pltpu.VMEM((4 * B,), jnp.int32),
            pltpu.VMEM((B, 256), jnp.float32),
            pltpu.VMEM((4 * B,), jnp.float32),
            pltpu.VMEM((256,), jnp.int32),
            pltpu.VMEM_SHARED((16 * (ES + SEG),), jnp.int32),
            pltpu.VMEM_SHARED((256,), jnp.int32),
            pltpu.SemaphoreType.DMA,
            pltpu.SemaphoreType.DMA,
            pltpu.SemaphoreType.DMA,
        ],
    )
    return f(xs, src, dst, asf, adf, amax16)


def _att_mat(att):
    return (att[:, :, None] * jnp.eye(HEADS, dtype=jnp.float32)[:, None, :]).reshape(HIDDEN, HEADS)


def _gat_sc(xs_feat, xd_feat, ei, p, n_dst):
    z = jnp.zeros((HIDDEN,), jnp.float32)
    z4 = jnp.zeros((HEADS,), jnp.float32)
    xs = _matmul_bias(xs_feat, p['Ws'], z)
    a_s = _matmul_bias(xs, _att_mat(p['att_s']), z4)
    a_d = _matmul_bias(xd_feat, p['Wd'] @ _att_mat(p['att_d']), z4)
    src, dst = ei[0].astype(jnp.int32), ei[1].astype(jnp.int32)
    amax = jnp.max(a_s, axis=0)

    src_p = jnp.pad(src, (0, E_PAD - E))
    dst_p = jnp.pad(dst, (0, E_PAD - E), constant_values=-1)
    asf = a_s.reshape(-1)
    adf = a_d.reshape(-1)
    amax16 = jnp.pad(amax, (0, 16 - HEADS))

    nch = -(-((n_dst + 1) // 2) // R)
    outs, _stg, _stgd = _sc_aggregate(xs, src_p, dst_p, asf, adf, amax16, nch)
    num = outs[:n_dst, :HIDDEN]
    den = outs[:n_dst, HIDDEN:HIDDEN + HEADS]
    out = num.reshape(n_dst, HEADS, DH) / (den[:, :, None] + 1e-16)
    return out.reshape(n_dst, HIDDEN) + p['b']


def _mm_kernel(x_ref, w_ref, b_ref, o_ref):
    o_ref[...] = jnp.dot(x_ref[...], w_ref[...],
                         preferred_element_type=jnp.float32) + b_ref[...]


def _matmul_bias(x, w, b, block_m=2048):
    m, k = x.shape
    n = w.shape[1]
    n_pad = (n + 127) // 128 * 128
    m_pad = (m + block_m - 1) // block_m * block_m
    if n_pad != n:
        w = jnp.pad(w, ((0, 0), (0, n_pad - n)))
        b = jnp.pad(b, ((0, n_pad - n),))
    if m_pad != m:
        x = jnp.pad(x, ((0, m_pad - m), (0, 0)))
    out = pl.pallas_call(
        _mm_kernel,
        grid=(m_pad // block_m,),
        in_specs=[
            pl.BlockSpec((block_m, k), lambda i: (i, 0)),
            pl.BlockSpec((k, n_pad), lambda i: (0, 0)),
            pl.BlockSpec((n_pad,), lambda i: (0,)),
        ],
        out_specs=pl.BlockSpec((block_m, n_pad), lambda i: (i, 0)),
        out_shape=jax.ShapeDtypeStruct((m_pad, n_pad), jnp.float32),
    )(x, w, b)
    return out[:m, :n]


def kernel(x_paper, edge_writes, edge_cites, edge_has_topic, edge_affiliated_with, edge_employs, edge_rev_topic, edge_written_by, edge_referenced_by, params):
    edges = {'writes': edge_writes, 'cites': edge_cites, 'has_topic': edge_has_topic, 'affiliated_with': edge_affiliated_with, 'employs': edge_employs, 'rev_topic': edge_rev_topic, 'written_by': edge_written_by, 'referenced_by': edge_referenced_by}
    h = {}
    for nt in NUM_NODES:
        feat = x_paper if nt == 'paper' else params['emb'][nt]
        h[nt] = _matmul_bias(feat, params['proj'][nt]['W'], params['proj'][nt]['b'])

    def hetero_layer(hin, conv):
        out = {nt: jnp.zeros((NUM_NODES[nt], HIDDEN), jnp.float32) for nt in NUM_NODES}
        for (s, r, d) in EDGE_TYPES:
            out[d] = out[d] + _gat_sc(hin[s], hin[d], edges[r], conv[r], NUM_NODES[d])
        return out

    h1 = {nt: jax.nn.relu(v) for nt, v in hetero_layer(h, params['conv1']).items()}
    h2 = hetero_layer(h1, params['conv2'])
    return _matmul_bias(h2['paper'], params['cls']['W'], params['cls']['b'])


# speedup vs baseline: 1.0176x; 1.0176x over previous
import functools

import jax
import jax.numpy as jnp
from jax import lax
from jax.experimental import pallas as pl
from jax.experimental.pallas import tpu as pltpu
from jax.experimental.pallas import tpu_sc as plsc

NUM_NODES = {'paper': 100000, 'author': 100000, 'institution': 8740, 'field_of_study': 59965}
EDGE_TYPES = [('author', 'writes', 'paper'), ('paper', 'cites', 'paper'), ('paper', 'has_topic', 'field_of_study'), ('author', 'affiliated_with', 'institution'), ('institution', 'employs', 'author'), ('field_of_study', 'rev_topic', 'paper'), ('paper', 'written_by', 'author'), ('paper', 'referenced_by', 'paper')]
HIDDEN = 256
HEADS = 4
DH = HIDDEN // HEADS
E = 100000

NTILES = 16
NCORES = 2
ES = 6256
E_PAD = NTILES * ES
NB = ES // 16
B = 48
R = 2560
SR = R // NTILES
SEG = 512
EST = ES + B
NSLAB = NCORES * NTILES * EST + 8


def _seg_kernel(nch, xs_hbm, src_hbm, dst_hbm, asf_hbm, adf_hbm, amax_hbm,
                out_hbm, stage_hbm, stgd_hbm,
                srcb, dstb, clid,
                ibatch, mbatch, gbuf, sbuf, dbuf, asq, adq, sdidx, ddidx,
                amaxb, cntw, acc,
                mbuf, cbuf, gidx, dgidx, abuf, dabuf, cntb,
                meta_sh, cnt_sh, gsem, asem, bsem):
    c = lax.axis_index("c")
    s = lax.axis_index("s")
    h_pad = nch * R
    sbase = (c * NTILES + s) * EST

    pltpu.sync_copy(src_hbm.at[pl.ds(pl.multiple_of(s * ES, 8), ES)], srcb.at[pl.ds(0, ES)])
    pltpu.sync_copy(dst_hbm.at[pl.ds(pl.multiple_of(s * ES, 8), ES)], dstb)
    pltpu.sync_copy(amax_hbm, amaxb)

    zv = jnp.zeros((16,), jnp.float32)
    zi = jnp.zeros((16,), jnp.int32)
    srcb[pl.ds(ES, 16)] = zi

    lanes = lax.iota(jnp.int32, 16)
    qh = jnp.bitwise_and(lanes, 3)
    qe = jnp.right_shift(lanes, 2)
    amaxq = plsc.load_gather(amaxb, [qh])

    def chunk_body(ci, _):
        lo = c * h_pad + ci * R

        def comp_body(b, cnt):
            off = b * 16
            dl = dstb[pl.ds(off, 16)] - lo
            m = (dl >= 0) & (dl < R)
            cum = plsc.cumsum(jnp.where(m, 1, 0))
            plsc.store_scatter(clid, [cnt + cum - 1], off + lanes, mask=m)
            return cnt + jnp.max(cum)

        cnt = lax.fori_loop(0, NB, comp_body, jnp.int32(0))
        padv = jnp.full((16,), ES, jnp.int32)
        for j in range(B // 16):
            clid[pl.ds(cnt + j * 16, 16)] = padv

        nbat = (cnt + (B - 1)) // B

        def batch_body(k, _):
            base = k * B
            for j in range(B // 16):
                lidv = clid[pl.ds(base + j * 16, 16)]
                ibatch[pl.ds(j * 16, 16)] = plsc.load_gather(srcb, [lidv])
                ov = plsc.load_gather(dstb, [jnp.minimum(lidv, ES - 1)]) - lo
                mbatch[pl.ds(j * 16, 16)] = (base + j * 16 + lanes) * 4096 + ov
            for j2 in range(B // 4):
                lidq = plsc.load_gather(clid, [base + j2 * 4 + qe])
                srcq = plsc.load_gather(srcb, [jnp.minimum(lidq, ES)])
                sdidx[pl.ds(j2 * 16, 16)] = srcq * 4 + qh
                dstq = plsc.load_gather(dstb, [jnp.minimum(lidq, ES - 1)])
                ddidx[pl.ds(j2 * 16, 16)] = dstq * 4 + qh
            cp1 = pltpu.async_copy(asf_hbm.at[sdidx], asq, asem)
            cp2 = pltpu.async_copy(adf_hbm.at[ddidx], adq, bsem)
            cp3 = pltpu.async_copy(xs_hbm.at[ibatch], gbuf, gsem)
            cp1.wait()
            cp2.wait()
            for j2 in range(B * 4 // 16):
                av = asq[pl.ds(j2 * 16, 16)] + adq[pl.ds(j2 * 16, 16)]
                ev_ = jnp.maximum(av, 0.0) + 0.2 * jnp.minimum(av, 0.0)
                mv = adq[pl.ds(j2 * 16, 16)] + amaxq
                mb_ = jnp.maximum(mv, 0.0) + 0.2 * jnp.minimum(mv, 0.0)
                dbuf[pl.ds(j2 * 16, 16)] = jnp.exp(ev_ - mb_)
            cp3.wait()

            def edge_body(i, _):
                for r in (2 * i, 2 * i + 1):
                    idxv = jnp.full((16,), 4 * r, jnp.int32)
                    ev = [plsc.load_gather(dbuf, [idxv + h]) for h in range(4)]
                    for j in range(16):
                        sbuf[r, pl.ds(j * 16, 16)] = gbuf[r, pl.ds(j * 16, 16)] * ev[j // 4]
                return 0

            lax.fori_loop(0, B // 2, edge_body, 0)
            pltpu.sync_copy(sbuf, stage_hbm.at[pl.ds(pl.multiple_of(sbase + base, 8), B)])
            pltpu.sync_copy(dbuf, stgd_hbm.at[pl.ds(pl.multiple_of(4 * (sbase + base), 8), 4 * B)])
            pltpu.sync_copy(mbatch, meta_sh.at[pl.ds(pl.multiple_of(s * (ES + SEG) + base, 8), B)])
            return 0

        lax.fori_loop(0, nbat, batch_body, 0)
        cntw[pl.ds(0, 16)] = jnp.full((16,), cnt, jnp.int32)
        pltpu.sync_copy(cntw, cnt_sh.at[pl.ds(pl.multiple_of(s * 16, 8), 16)])
        plsc.subcore_barrier()

        own_lo = s * SR

        def zrow_body(r2, _):
            for j in range(17):
                acc[r2, pl.ds(j * 16, 16)] = zv
            return 0

        lax.fori_loop(0, SR, zrow_body, 0)
        pltpu.sync_copy(cnt_sh, cntb)
        wtail = jnp.full((16,), (s + 1) * SR, jnp.int32)

        def src_tile_body(s2, _):
            cnt2 = jnp.max(cntb[pl.ds(s2 * 16, 16)])
            gb = (c * NTILES + s2) * EST

            def seg_cond(st):
                return st < cnt2

            def seg_body(st):
                pltpu.sync_copy(meta_sh.at[pl.ds(pl.multiple_of(s2 * (ES + SEG) + st, 8), SEG)], mbuf.at[pl.ds(0, SEG)])
                n = jnp.minimum(cnt2 - st, SEG)

                def blk(b2, cc):
                    w = mbuf[pl.ds(b2 * 16, 16)]
                    dl = jnp.bitwise_and(w, 4095)
                    ok = ((b2 * 16 + lanes) < n) & (dl >= own_lo) & (dl < own_lo + SR)
                    cum = plsc.cumsum(jnp.where(ok, 1, 0))
                    plsc.store_scatter(cbuf, [cc + cum - 1], w, mask=ok)
                    return cc + jnp.max(cum)

                cc = lax.fori_loop(0, SEG // 16, blk, jnp.int32(0))
                for j in range(B // 16):
                    cbuf[pl.ds(cc + j * 16, 16)] = wtail
                nb2 = (cc + (B - 1)) // B

                def b2_body(k2, _):
                    for j in range(B // 16):
                        wv = cbuf[pl.ds(k2 * B + j * 16, 16)]
                        gidx[pl.ds(j * 16, 16)] = gb + jnp.right_shift(wv, 12)
                    for j2 in range(B // 4):
                        wq = plsc.load_gather(
                            cbuf, [k2 * B + j2 * 4 + jnp.right_shift(lanes, 2)])
                        dgidx[pl.ds(j2 * 16, 16)] = (
                            (gb + jnp.right_shift(wq, 12)) * 4 + jnp.bitwise_and(lanes, 3))
                    pltpu.async_copy(stage_hbm.at[gidx], abuf, asem).wait()
                    pltpu.async_copy(stgd_hbm.at[dgidx], dabuf, asem).wait()

                    def erow(i3, _):
                        for r3 in (2 * i3, 2 * i3 + 1):
                            wv = plsc.load_gather(cbuf, [jnp.full((16,), k2 * B + r3, jnp.int32)])
                            dlr = jnp.max(jnp.bitwise_and(wv, 4095)) - own_lo
                            for j in range(16):
                                acc[dlr, pl.ds(j * 16, 16)] = (
                                    acc[dlr, pl.ds(j * 16, 16)] + abuf[r3, pl.ds(j * 16, 16)])
                            dg = plsc.load_gather(dabuf, [r3 * 4 + jnp.minimum(lanes, 3)])
                            dv = jnp.where(lanes < 4, dg, zv)
                            acc[dlr, pl.ds(256, 16)] = acc[dlr, pl.ds(256, 16)] + dv
                        return 0

                    lax.fori_loop(0, B // 2, erow, 0)
                    return 0

                lax.fori_loop(0, nb2, b2_body, 0)
                return st + SEG

            lax.while_loop(seg_cond, seg_body, jnp.int32(0))
            return 0

        lax.fori_loop(0, NTILES, src_tile_body, 0)

        pltpu.sync_copy(acc.at[pl.ds(0, SR)], out_hbm.at[pl.ds(pl.multiple_of(lo + own_lo, 8), SR)])
        plsc.subcore_barrier()
        return 0

    lax.fori_loop(0, nch, chunk_body, 0)


@functools.partial(jax.jit, static_argnums=(6,))
def _sc_aggregate(xs, src, dst, asf, adf, amax16, nch):
    h_pad = nch * R
    mesh = plsc.VectorSubcoreMesh(core_axis_name="c", subcore_axis_name="s")
    f = pl.kernel(
        functools.partial(_seg_kernel, nch),
        compiler_params=pltpu.CompilerParams(needs_layout_passes=False),
        out_type=(
            jax.ShapeDtypeStruct((2 * h_pad, 272), jnp.float32),
            jax.ShapeDtypeStruct((NSLAB, 256), jnp.float32),
            jax.ShapeDtypeStruct((4 * NSLAB,), jnp.float32),
        ),
        mesh=mesh,
        scratch_types=[
            pltpu.VMEM((ES + 16,), jnp.int32),
            pltpu.VMEM((ES,), jnp.int32),
            pltpu.VMEM((ES + B,), jnp.int32),
            pltpu.VMEM((B,), jnp.int32),
            pltpu.VMEM((B,), jnp.int32),
            pltpu.VMEM((B, 256), jnp.float32),
            pltpu.VMEM((B, 256), jnp.float32),
            pltpu.VMEM((4 * B,), jnp.float32),
            pltpu.VMEM((4 * B,), jnp.float32),
            pltpu.VMEM((4 * B,), jnp.float32),
            pltpu.VMEM((4 * B,), jnp.int32),
            pltpu.VMEM((4 * B,), jnp.int32),
            pltpu.VMEM((16,), jnp.float32),
            pltpu.VMEM((16,), jnp.int32),
            pltpu.VMEM((SR + 8, 272), jnp.float32),
            pltpu.VMEM((SEG + 16,), jnp.int32),
            pltpu.VMEM((SEG + B + 16,), jnp.int32),
            pltpu.VMEM((B,), jnp.int32),
            pltpu.VMEM((4 * B,), jnp.int32),
            pltpu.VMEM((B, 256), jnp.float32),
            pltpu.VMEM((4 * B,), jnp.float32),
            pltpu.VMEM((256,), jnp.int32),
            pltpu.VMEM_SHARED((16 * (ES + SEG),), jnp.int32),
            pltpu.VMEM_SHARED((256,), jnp.int32),
            pltpu.SemaphoreType.DMA,
            pltpu.SemaphoreType.DMA,
            pltpu.SemaphoreType.DMA,
        ],
    )
    return f(xs, src, dst, asf, adf, amax16)


def _att_mat(att):
    return (att[:, :, None] * jnp.eye(HEADS, dtype=jnp.float32)[:, None, :]).reshape(HIDDEN, HEADS)


def _gat_sc(xs_feat, xd_feat, ei, p, n_dst):
    z = jnp.zeros((HIDDEN,), jnp.float32)
    z4 = jnp.zeros((HEADS,), jnp.float32)
    xs = _matmul_bias(xs_feat, p['Ws'], z)
    a_s = _matmul_bias(xs, _att_mat(p['att_s']), z4)
    a_d = _matmul_bias(xd_feat, p['Wd'] @ _att_mat(p['att_d']), z4)
    src, dst = ei[0].astype(jnp.int32), ei[1].astype(jnp.int32)
    amax = jnp.max(a_s, axis=0)

    src_p = jnp.pad(src, (0, E_PAD - E))
    dst_p = jnp.pad(dst, (0, E_PAD - E), constant_values=-1)
    asf = a_s.reshape(-1)
    adf = a_d.reshape(-1)
    amax16 = jnp.pad(amax, (0, 16 - HEADS))

    nch = -(-((n_dst + 1) // 2) // R)
    outs, _stg, _stgd = _sc_aggregate(xs, src_p, dst_p, asf, adf, amax16, nch)
    num = outs[:n_dst, :HIDDEN]
    den = outs[:n_dst, HIDDEN:HIDDEN + HEADS]
    out = num.reshape(n_dst, HEADS, DH) / (den[:, :, None] + 1e-16)
    return out.reshape(n_dst, HIDDEN) + p['b']


def _mm_kernel(x_ref, w_ref, b_ref, o_ref):
    o_ref[...] = jnp.dot(x_ref[...], w_ref[...],
                         preferred_element_type=jnp.float32) + b_ref[...]


def _matmul_bias(x, w, b, block_m=2048):
    m, k = x.shape
    n = w.shape[1]
    n_pad = (n + 127) // 128 * 128
    m_pad = (m + block_m - 1) // block_m * block_m
    if n_pad != n:
        w = jnp.pad(w, ((0, 0), (0, n_pad - n)))
        b = jnp.pad(b, ((0, n_pad - n),))
    if m_pad != m:
        x = jnp.pad(x, ((0, m_pad - m), (0, 0)))
    out = pl.pallas_call(
        _mm_kernel,
        grid=(m_pad // block_m,),
        in_specs=[
            pl.BlockSpec((block_m, k), lambda i: (i, 0)),
            pl.BlockSpec((k, n_pad), lambda i: (0, 0)),
            pl.BlockSpec((n_pad,), lambda i: (0,)),
        ],
        out_specs=pl.BlockSpec((block_m, n_pad), lambda i: (i, 0)),
        out_shape=jax.ShapeDtypeStruct((m_pad, n_pad), jnp.float32),
    )(x, w, b)
    return out[:m, :n]


def kernel(x_paper, edge_writes, edge_cites, edge_has_topic, edge_affiliated_with, edge_employs, edge_rev_topic, edge_written_by, edge_referenced_by, params):
    edges = {'writes': edge_writes, 'cites': edge_cites, 'has_topic': edge_has_topic, 'affiliated_with': edge_affiliated_with, 'employs': edge_employs, 'rev_topic': edge_rev_topic, 'written_by': edge_written_by, 'referenced_by': edge_referenced_by}
    h = {}
    for nt in NUM_NODES:
        feat = x_paper if nt == 'paper' else params['emb'][nt]
        h[nt] = _matmul_bias(feat, params['proj'][nt]['W'], params['proj'][nt]['b'])

    def hetero_layer(hin, conv):
        out = {nt: jnp.zeros((NUM_NODES[nt], HIDDEN), jnp.float32) for nt in NUM_NODES}
        for (s, r, d) in EDGE_TYPES:
            out[d] = out[d] + _gat_sc(hin[s], hin[d], edges[r], conv[r], NUM_NODES[d])
        return out

    h1 = {nt: jax.nn.relu(v) for nt, v in hetero_layer(h, params['conv1']).items()}
    h2 = hetero_layer(h1, params['conv2'])
    return _matmul_bias(h2['paper'], params['cls']['W'], params['cls']['b'])
